# baseline (speedup 1.0000x reference)
import functools
import jax
import jax.numpy as jnp
from jax import lax
from jax.experimental import pallas as pl
from jax.experimental.pallas import tpu as pltpu
from jax.experimental.pallas import tpu_sc as plsc

B, T, S, V, DIM = 1024, 200, 50, 1000, 16
TP = 208
SP = 56
NW = 32
NG = 2
NGRP = B // 16


def _sc_body(encp_h, spl_h, w_h, bias_h, out_h,
             enc16_v, spl16_v, tbl_v, e16_v, pe_v, out16_v, bias_v,
             dupt_v, dupp_v, gsem):
    wid = lax.axis_index("s") * 2 + lax.axis_index("c")
    iota = lax.iota(jnp.int32, 16)
    zeros16i = jnp.zeros((16,), jnp.int32)

    pltpu.sync_copy(bias_h, bias_v)
    bias_row = bias_v[pl.ds(0, 16)]

    def clr(r, carry):
        tbl_v[pl.ds(r * 16, 16)] = zeros16i
        return carry
    lax.fori_loop(0, V, clr, 0)

    lanes_base = iota * TP

    for g in range(NG):
        G = wid * NG + g
        pltpu.sync_copy(encp_h.at[G], enc16_v)
        pltpu.sync_copy(spl_h.at[G], spl16_v)
        copies = []
        for l0 in range(16):
            for cc in range(2):
                copies.append(pltpu.async_copy(
                    w_h.at[enc16_v.at[pl.ds(l0 * TP + cc * 104, 104)]],
                    e16_v.at[pl.ds(l0 * TP + cc * 104, 104)],
                    gsem))

        tag0 = (g + 1) * 256

        def aloop(t, cnt):
            v = plsc.load_gather(enc16_v, [lanes_base + t])
            fidx = v * 16 + iota
            lp = plsc.load_gather(tbl_v, [fidx])
            plsc.store_scatter(tbl_v, [fidx],
                               jnp.full((16,), tag0 + t, jnp.int32))
            valid = lax.shift_right_logical(lp, 8) == (g + 1)
            prevt = lp & 255
            didx = lanes_base + cnt
            plsc.store_scatter(dupt_v, [didx],
                               jnp.full((16,), t, jnp.int32), mask=valid)
            plsc.store_scatter(dupp_v, [didx], prevt, mask=valid)
            return cnt + valid.astype(jnp.int32)

        cnt_vec = lax.fori_loop(0, T, aloop, zeros16i)
        for c in copies:
            c.wait()

        def lane_body(l, carry):
            cntl = jnp.sum(jnp.where(iota == l, cnt_vec, 0))
            erow0 = l * TP
            pe_v[pl.ds(0, 16)] = jnp.zeros((16,), jnp.float32)

            def ploop(t, pe):
                pe = pe + e16_v[erow0 + t]
                pe_v[pl.ds((t + 1) * 16, 16)] = pe
                return pe

            lax.fori_loop(0, T, ploop, jnp.zeros((16,), jnp.float32))

            sbase = l * (2 * SP)
            ndc = (cntl + 15) // 16

            def sloop(s, scarry):
                c16 = (s // 16) * 16
                lane = s - c16
                iv = spl16_v[pl.ds(sbase + c16, 16)]
                jv = spl16_v[pl.ds(sbase + SP + c16, 16)]
                i = jnp.sum(jnp.where(iota == lane, iv, 0))
                j = jnp.sum(jnp.where(iota == lane, jv, 0))
                base = (pe_v[pl.ds(j * 16, 16)]
                        - pe_v[pl.ds(i * 16, 16)] + bias_row)

                def corr(d, b):
                    dbase = d * 16
                    td = plsc.load_gather(dupt_v, [erow0 + dbase + iota])
                    pd = plsc.load_gather(dupp_v, [erow0 + dbase + iota])
                    m = ((dbase + iota) < cntl) & (td < j) & (pd >= i)
                    npop = plsc.all_reduce_population_count(m)[0]

                    def inner(k, bm):
                        bb, mi = bm
                        lsel = plsc.all_reduce_ffs(mi != 0)[0]
                        tdl = jnp.sum(jnp.where(iota == lsel, td, 0))
                        bb = bb - e16_v[erow0 + tdl]
                        mi = mi * (iota != lsel).astype(jnp.int32)
                        return (bb, mi)

                    b, _ = lax.fori_loop(0, npop, inner,
                                         (b, m.astype(jnp.int32)))
                    return b

                base = lax.fori_loop(0, ndc, corr, base)
                out16_v[pl.ds(l * (S * DIM) + s * 16, 16)] = base
                return scarry

            lax.fori_loop(0, S, sloop, 0)
            return carry

        lax.fori_loop(0, 16, lane_body, 0)
        pltpu.sync_copy(out16_v, out_h.at[G])


def kernel(word_encs, span_idxs, W, bias):
    enc = word_encs.astype(jnp.int32)
    enc_pad = jnp.zeros((B, TP), jnp.int32).at[:, :T].set(enc)
    lo = span_idxs[:, :, 0].astype(jnp.int32)
    hi = span_idxs[:, :, 1].astype(jnp.int32)
    spl = jnp.zeros((B, 2 * SP), jnp.int32)
    spl = spl.at[:, :S].set(lo).at[:, SP:SP + S].set(hi)
    encp_h = enc_pad.reshape(NGRP, 16 * TP)
    spl_h = spl.reshape(NGRP, 16 * 2 * SP)

    mesh = plsc.VectorSubcoreMesh(core_axis_name="c", subcore_axis_name="s")
    f = functools.partial(
        pl.kernel,
        out_type=jax.ShapeDtypeStruct((NGRP, 16 * S * DIM), jnp.float32),
        mesh=mesh,
        compiler_params=pltpu.CompilerParams(
            needs_layout_passes=False, use_tc_tiling_on_sc=False),
        scratch_types=[
            pltpu.VMEM((16 * TP,), jnp.int32),
            pltpu.VMEM((16 * 2 * SP,), jnp.int32),
            pltpu.VMEM((V * 16,), jnp.int32),
            pltpu.VMEM((16 * TP, DIM), jnp.float32),
            pltpu.VMEM(((T + 8) * 16,), jnp.float32),
            pltpu.VMEM((16 * S * DIM,), jnp.float32),
            pltpu.VMEM((16,), jnp.float32),
            pltpu.VMEM((16 * TP,), jnp.int32),
            pltpu.VMEM((16 * TP,), jnp.int32),
            pltpu.SemaphoreType.DMA,
        ],
    )(_sc_body)
    out = f(encp_h, spl_h, W.astype(jnp.float32), bias.astype(jnp.float32))
    return out.reshape(B, S, DIM)


# speedup vs baseline: 1.3155x; 1.3155x over previous
import functools
import jax
import jax.numpy as jnp
from jax import lax
from jax.experimental import pallas as pl
from jax.experimental.pallas import tpu as pltpu
from jax.experimental.pallas import tpu_sc as plsc

B, T, S, V, DIM = 1024, 200, 50, 1000, 16
TP = 208
NG = 2
NGRP = B // 16
BB = 8


def _sc_body(encp_h, w_h, e_h, prev_h, enc16_v, tbl_v, e16_v, prev16_v, gsem):
    wid = lax.axis_index("s") * 2 + lax.axis_index("c")
    iota = lax.iota(jnp.int32, 16)
    zeros16i = jnp.zeros((16,), jnp.int32)

    def clr(r, carry):
        tbl_v[pl.ds(r * 16, 16)] = zeros16i
        return carry
    lax.fori_loop(0, V, clr, 0)

    lanes_base = iota * TP

    for g in range(NG):
        G = wid * NG + g
        pltpu.sync_copy(encp_h.at[G], enc16_v)
        copies = []
        for l0 in range(16):
            for cc in range(2):
                copies.append(pltpu.async_copy(
                    w_h.at[enc16_v.at[pl.ds(l0 * TP + cc * 104, 104)]],
                    e16_v.at[pl.ds(l0 * TP + cc * 104, 104)],
                    gsem))

        tag0 = (g + 1) * 256

        def aloop(t, carry):
            v = plsc.load_gather(enc16_v, [lanes_base + t])
            fidx = v * 16 + iota
            lp = plsc.load_gather(tbl_v, [fidx])
            plsc.store_scatter(tbl_v, [fidx],
                               jnp.full((16,), tag0 + t, jnp.int32))
            valid = lax.shift_right_logical(lp, 8) == (g + 1)
            prevt = jnp.where(valid, lp & 255, jnp.full((16,), -1, jnp.int32))
            plsc.store_scatter(prev16_v, [lanes_base + t], prevt)
            return carry

        lax.fori_loop(0, T, aloop, 0)
        for c in copies:
            c.wait()
        pltpu.sync_copy(e16_v, e_h.at[G])
        pltpu.sync_copy(prev16_v, prev_h.at[G])


def _tc_kernel(lo_ref, hi_ref, prev_ref, e_ref, bias_ref, out_ref):
    lo = lo_ref[...]
    hi = hi_ref[...]
    prev = prev_ref[...]
    pos = lax.broadcasted_iota(jnp.int32, (BB, S, TP), 2)
    lob = lo[:, :, None]
    mask = ((pos >= lob) & (pos < hi[:, :, None])
            & (prev[:, None, :] < lob)).astype(jnp.float32)
    out = lax.dot_general(
        mask, e_ref[...],
        dimension_numbers=(((2,), (1,)), ((0,), (0,))),
        preferred_element_type=jnp.float32,
    )
    out_ref[...] = out + bias_ref[...][None, None, :]


def kernel(word_encs, span_idxs, W, bias):
    enc = word_encs.astype(jnp.int32)
    enc_pad = jnp.zeros((B, TP), jnp.int32).at[:, :T].set(enc)
    encp_h = enc_pad.reshape(NGRP, 16 * TP)

    mesh = plsc.VectorSubcoreMesh(core_axis_name="c", subcore_axis_name="s")
    sc = functools.partial(
        pl.kernel,
        out_type=(
            jax.ShapeDtypeStruct((NGRP, 16 * TP, DIM), jnp.float32),
            jax.ShapeDtypeStruct((NGRP, 16 * TP), jnp.int32),
        ),
        mesh=mesh,
        compiler_params=pltpu.CompilerParams(
            needs_layout_passes=False, use_tc_tiling_on_sc=False),
        scratch_types=[
            pltpu.VMEM((16 * TP,), jnp.int32),
            pltpu.VMEM((V * 16,), jnp.int32),
            pltpu.VMEM((16 * TP, DIM), jnp.float32),
            pltpu.VMEM((16 * TP,), jnp.int32),
            pltpu.SemaphoreType.DMA,
        ],
    )(_sc_body)
    e_rows, prev = sc(encp_h, W.astype(jnp.float32))
    e_rows = e_rows.reshape(B, TP, DIM)
    prev = prev.reshape(B, TP)

    lo = span_idxs[:, :, 0].astype(jnp.int32)
    hi = span_idxs[:, :, 1].astype(jnp.int32)
    return pl.pallas_call(
        _tc_kernel,
        grid=(B // BB,),
        in_specs=[
            pl.BlockSpec((BB, S), lambda g: (g, 0)),
            pl.BlockSpec((BB, S), lambda g: (g, 0)),
            pl.BlockSpec((BB, TP), lambda g: (g, 0)),
            pl.BlockSpec((BB, TP, DIM), lambda g: (g, 0, 0)),
            pl.BlockSpec((DIM,), lambda g: (0,)),
        ],
        out_specs=pl.BlockSpec((BB, S, DIM), lambda g: (g, 0, 0)),
        out_shape=jax.ShapeDtypeStruct((B, S, DIM), jnp.float32),
    )(lo, hi, prev, e_rows, bias.astype(jnp.float32))


# speedup vs baseline: 1.3162x; 1.0005x over previous
import functools
import jax
import jax.numpy as jnp
from jax import lax
from jax.experimental import pallas as pl
from jax.experimental.pallas import tpu as pltpu
from jax.experimental.pallas import tpu_sc as plsc

B, T, S, V, DIM = 1024, 200, 50, 1000, 16
TP = 208
NG = 2
NGRP = B // 16
BB = 8


def _sc_body(encp_h, w_h, e_h, prev_h, enc16_v, tbl_v, e16_v, prev16_v, gsem):
    wid = lax.axis_index("s") * 2 + lax.axis_index("c")
    iota = lax.iota(jnp.int32, 16)
    zeros16i = jnp.zeros((16,), jnp.int32)

    def clr(r, carry):
        tbl_v[pl.ds(r * 16, 16)] = zeros16i
        return carry
    lax.fori_loop(0, V, clr, 0)

    lanes_base = iota * TP

    for g in range(NG):
        G = wid * NG + g
        pltpu.sync_copy(encp_h.at[G], enc16_v)
        copies = []
        for l0 in range(16):
            for cc in range(2):
                copies.append(pltpu.async_copy(
                    w_h.at[enc16_v.at[pl.ds(l0 * TP + cc * 104, 104)]],
                    e16_v.at[pl.ds(l0 * TP + cc * 104, 104)],
                    gsem))

        tag0 = (g + 1) * 256

        def aloop(t, carry):
            v = plsc.load_gather(enc16_v, [lanes_base + t])
            fidx = v * 16 + iota
            lp = plsc.load_gather(tbl_v, [fidx])
            plsc.store_scatter(tbl_v, [fidx],
                               jnp.full((16,), tag0 + t, jnp.int32))
            valid = lax.shift_right_logical(lp, 8) == (g + 1)
            prevt = jnp.where(valid, lp & 255, jnp.full((16,), -1, jnp.int32))
            plsc.store_scatter(prev16_v, [lanes_base + t], prevt)
            return carry

        lax.fori_loop(0, T, aloop, 0)
        for c in copies:
            c.wait()
        pltpu.sync_copy(e16_v, e_h.at[G])
        pltpu.sync_copy(prev16_v, prev_h.at[G])


def _tc_kernel(lo_ref, hi_ref, prev_ref, e_ref, bias_ref, out_ref):
    lo = lo_ref[...]
    hi = hi_ref[...]
    prev = prev_ref[...]
    pos = lax.broadcasted_iota(jnp.int32, (BB, S, TP), 2)
    lob = lo[:, :, None]
    mask = ((pos >= lob) & (pos < hi[:, :, None])
            & (prev[:, None, :] < lob)).astype(jnp.float32)
    out = lax.dot_general(
        mask, e_ref[...],
        dimension_numbers=(((2,), (1,)), ((0,), (0,))),
        preferred_element_type=jnp.float32,
    )
    out_ref[...] = out + bias_ref[...][None, None, :]


def kernel(word_encs, span_idxs, W, bias):
    enc = word_encs.astype(jnp.int32)
    enc_pad = jnp.zeros((B, TP), jnp.int32).at[:, :T].set(enc)
    encp_h = enc_pad.reshape(NGRP, 16 * TP)

    mesh = plsc.VectorSubcoreMesh(core_axis_name="c", subcore_axis_name="s")
    sc = functools.partial(
        pl.kernel,
        out_type=(
            jax.ShapeDtypeStruct((NGRP, 16 * TP, DIM), jnp.float32),
            jax.ShapeDtypeStruct((NGRP, 16 * TP), jnp.int32),
        ),
        mesh=mesh,
        compiler_params=pltpu.CompilerParams(
            needs_layout_passes=False, use_tc_tiling_on_sc=False,
            skip_device_barrier=True),
        scratch_types=[
            pltpu.VMEM((16 * TP,), jnp.int32),
            pltpu.VMEM((V * 16,), jnp.int32),
            pltpu.VMEM((16 * TP, DIM), jnp.float32),
            pltpu.VMEM((16 * TP,), jnp.int32),
            pltpu.SemaphoreType.DMA,
        ],
    )(_sc_body)
    e_rows, prev = sc(encp_h, W.astype(jnp.float32))
    e_rows = e_rows.reshape(B, TP, DIM)
    prev = prev.reshape(B, TP)

    lo = span_idxs[:, :, 0].astype(jnp.int32)
    hi = span_idxs[:, :, 1].astype(jnp.int32)
    return pl.pallas_call(
        _tc_kernel,
        grid=(B // BB,),
        in_specs=[
            pl.BlockSpec((BB, S), lambda g: (g, 0)),
            pl.BlockSpec((BB, S), lambda g: (g, 0)),
            pl.BlockSpec((BB, TP), lambda g: (g, 0)),
            pl.BlockSpec((BB, TP, DIM), lambda g: (g, 0, 0)),
            pl.BlockSpec((DIM,), lambda g: (0,)),
        ],
        out_specs=pl.BlockSpec((BB, S, DIM), lambda g: (g, 0, 0)),
        out_shape=jax.ShapeDtypeStruct((B, S, DIM), jnp.float32),
    )(lo, hi, prev, e_rows, bias.astype(jnp.float32))
